# Initial kernel scaffold; baseline (speedup 1.0000x reference)
import jax
import jax.numpy as jnp
from jax.experimental import pallas as pl


def kernel(x, ln_scale, ln_bias, Wr, br, W1, b1, W2, b2):
    raise NotImplementedError("write your pallas kernel here")


# speedup vs baseline: 2.8711x; 2.8711x over previous
import functools

import jax
import jax.numpy as jnp
from jax import lax
from jax.experimental import pallas as pl
from jax.experimental.pallas import tpu as pltpu
from jax.experimental.pallas import tpu_sc as plsc

N = 8192
D = 768
H = 1536
E = 8
K = 2
EPS = 1e-05

A = N * K
BR = 256
RPAD = A + E * BR
NB = RPAD // BR
BN = 1024

NW = 32
ROWS_PER_W = RPAD // NW
GCHUNK = 96
TOK_PER_W = N // NW
TCHUNK = 64

_SQRT_HALF = 0.7071067811865476


def _stage1_body(x_ref, lns_ref, lnb_ref, wrt_ref, br_ref,
                 i0_ref, i1_ref, w0_ref, w1_ref):
    xb = x_ref[...]
    mu = jnp.mean(xb, axis=1, keepdims=True)
    var = jnp.mean((xb - mu) ** 2, axis=1, keepdims=True)
    xn = (xb - mu) * lax.rsqrt(var + EPS) * lns_ref[...] + lnb_ref[...]
    logits = jnp.dot(xn, wrt_ref[...], preferred_element_type=jnp.float32)
    logits = logits + br_ref[...]
    col = lax.broadcasted_iota(jnp.int32, logits.shape, 1)
    v0 = jnp.max(logits, axis=1)
    i0 = jnp.argmax(logits, axis=1).astype(jnp.int32)
    neg = jnp.float32(-3.0e38)
    masked = jnp.where(col == i0[:, None], neg, logits)
    v1 = jnp.max(masked, axis=1)
    i1 = jnp.argmax(masked, axis=1).astype(jnp.int32)
    g1 = 1.0 / (1.0 + jnp.exp(v0 - v1))
    i0_ref[...] = i0
    i1_ref[...] = i1
    w0_ref[...] = 1.0 - g1
    w1_ref[...] = g1


def _stage1(x, ln_scale, ln_bias, Wr, br):
    return pl.pallas_call(
        _stage1_body,
        grid=(N // BN,),
        in_specs=[
            pl.BlockSpec((BN, D), lambda b: (b, 0)),
            pl.BlockSpec((1, D), lambda b: (0, 0)),
            pl.BlockSpec((1, D), lambda b: (0, 0)),
            pl.BlockSpec((D, E), lambda b: (0, 0)),
            pl.BlockSpec((1, E), lambda b: (0, 0)),
        ],
        out_specs=[pl.BlockSpec((BN,), lambda b: (b,))] * 4,
        out_shape=[
            jax.ShapeDtypeStruct((N,), jnp.int32),
            jax.ShapeDtypeStruct((N,), jnp.int32),
            jax.ShapeDtypeStruct((N,), jnp.float32),
            jax.ShapeDtypeStruct((N,), jnp.float32),
        ],
    )(x, ln_scale.reshape(1, D), ln_bias.reshape(1, D), Wr.T, br.reshape(1, E))


def _routing_metadata(i0, i1, w0, w1):
    flat_e = jnp.stack([i0, i1], axis=1).reshape(A)
    flat_w = jnp.stack([w0, w1], axis=1).reshape(A)
    oh = (flat_e[:, None] == jnp.arange(E, dtype=jnp.int32)[None, :]).astype(jnp.int32)
    csum = jnp.cumsum(oh, axis=0)
    counts = csum[-1]
    cap = ((counts + BR - 1) // BR) * BR
    starts = jnp.concatenate([jnp.zeros((1,), jnp.int32), jnp.cumsum(cap)[:-1].astype(jnp.int32)])
    rank = jnp.sum(csum * oh, axis=1) - 1
    pos_flat = starts[flat_e] + rank
    gather_idx = jnp.zeros((RPAD,), jnp.int32).at[pos_flat].set(
        jnp.arange(A, dtype=jnp.int32) // K)
    wg = jnp.zeros((RPAD,), jnp.float32).at[pos_flat].set(flat_w)
    ends = (starts + cap).astype(jnp.int32)
    bstart = jnp.arange(NB, dtype=jnp.int32) * BR
    blk_expert = jnp.minimum(
        jnp.sum((bstart[:, None] >= ends[None, :]).astype(jnp.int32), axis=1),
        E - 1).astype(jnp.int32)
    pos2 = pos_flat.reshape(N, K)
    return gather_idx, wg.reshape(RPAD, 1), blk_expert, pos2[:, 0], pos2[:, 1]


@functools.lru_cache(maxsize=None)
def _sc_gather_fn():
    @functools.partial(
        pl.kernel,
        mesh=plsc.VectorSubcoreMesh(core_axis_name="c", subcore_axis_name="s"),
        out_type=jax.ShapeDtypeStruct((RPAD, D), jnp.float32),
        scratch_types=[
            pltpu.VMEM((GCHUNK,), jnp.int32),
            pltpu.VMEM((GCHUNK, D), jnp.float32),
            pltpu.SemaphoreType.DMA,
        ],
    )
    def _sc_gather(x_hbm, idx_hbm, out_hbm, idx_v, rows_v, sem):
        wid = lax.axis_index("s") * 2 + lax.axis_index("c")
        base = wid * ROWS_PER_W

        def chunk(c, carry):
            off = base + c * GCHUNK
            pltpu.sync_copy(idx_hbm.at[pl.ds(off, GCHUNK)], idx_v)
            pltpu.async_copy(x_hbm.at[idx_v], rows_v, sem).wait()
            pltpu.sync_copy(rows_v, out_hbm.at[pl.ds(off, GCHUNK)])
            return carry

        lax.fori_loop(0, ROWS_PER_W // GCHUNK, chunk, 0)

    return _sc_gather


def _ffn_body(be_ref, xg_ref, w1_ref, b1_ref, w2_ref, b2_ref, wg_ref,
              lns_ref, lnb_ref, out_ref):
    xb = xg_ref[...]
    mu = jnp.mean(xb, axis=1, keepdims=True)
    var = jnp.mean((xb - mu) ** 2, axis=1, keepdims=True)
    xn = (xb - mu) * lax.rsqrt(var + EPS) * lns_ref[...] + lnb_ref[...]
    h = lax.dot_general(xn, w1_ref[0], (((1,), (1,)), ((), ())),
                        preferred_element_type=jnp.float32)
    h = h + b1_ref[0]
    h = 0.5 * h * (1.0 + lax.erf(h * _SQRT_HALF))
    o = lax.dot_general(h, w2_ref[0], (((1,), (1,)), ((), ())),
                        preferred_element_type=jnp.float32)
    o = o + b2_ref[0]
    out_ref[...] = o * wg_ref[...]


def _ffn(blk_expert, Xg, W1, b1, W2, b2, wg, ln_scale, ln_bias):
    grid_spec = pltpu.PrefetchScalarGridSpec(
        num_scalar_prefetch=1,
        grid=(NB,),
        in_specs=[
            pl.BlockSpec((BR, D), lambda b, be: (b, 0)),
            pl.BlockSpec((1, H, D), lambda b, be: (be[b], 0, 0)),
            pl.BlockSpec((1, 1, H), lambda b, be: (be[b], 0, 0)),
            pl.BlockSpec((1, D, H), lambda b, be: (be[b], 0, 0)),
            pl.BlockSpec((1, 1, D), lambda b, be: (be[b], 0, 0)),
            pl.BlockSpec((BR, 1), lambda b, be: (b, 0)),
            pl.BlockSpec((1, D), lambda b, be: (0, 0)),
            pl.BlockSpec((1, D), lambda b, be: (0, 0)),
        ],
        out_specs=pl.BlockSpec((BR, D), lambda b, be: (b, 0)),
    )
    return pl.pallas_call(
        _ffn_body,
        grid_spec=grid_spec,
        out_shape=jax.ShapeDtypeStruct((RPAD, D), jnp.float32),
    )(blk_expert, Xg, W1, b1.reshape(E, 1, H), W2, b2.reshape(E, 1, D), wg,
      ln_scale.reshape(1, D), ln_bias.reshape(1, D))


@functools.lru_cache(maxsize=None)
def _sc_combine_fn():
    @functools.partial(
        pl.kernel,
        mesh=plsc.VectorSubcoreMesh(core_axis_name="c", subcore_axis_name="s"),
        out_type=jax.ShapeDtypeStruct((N, D), jnp.float32),
        scratch_types=[
            pltpu.VMEM((TCHUNK,), jnp.int32),
            pltpu.VMEM((TCHUNK,), jnp.int32),
            pltpu.VMEM((TCHUNK, D), jnp.float32),
            pltpu.VMEM((TCHUNK, D), jnp.float32),
            pltpu.SemaphoreType.DMA,
            pltpu.SemaphoreType.DMA,
        ],
    )
    def _sc_combine(rows_hbm, posa_hbm, posb_hbm, y_hbm,
                    ia_v, ib_v, bufa_v, bufb_v, sema, semb):
        wid = lax.axis_index("s") * 2 + lax.axis_index("c")
        base = wid * TOK_PER_W

        def chunk(c, carry):
            off = base + c * TCHUNK
            pltpu.sync_copy(posa_hbm.at[pl.ds(off, TCHUNK)], ia_v)
            pltpu.sync_copy(posb_hbm.at[pl.ds(off, TCHUNK)], ib_v)
            cpa = pltpu.async_copy(rows_hbm.at[ia_v], bufa_v, sema)
            cpb = pltpu.async_copy(rows_hbm.at[ib_v], bufb_v, semb)
            cpa.wait()
            cpb.wait()

            def add_row(i, carry2):
                for j in range(D // 16):
                    sl = pl.ds(j * 16, 16)
                    bufa_v[i, sl] = bufa_v[i, sl] + bufb_v[i, sl]
                return carry2

            lax.fori_loop(0, TCHUNK, add_row, 0)
            pltpu.sync_copy(bufa_v, y_hbm.at[pl.ds(off, TCHUNK)])
            return carry

        lax.fori_loop(0, TOK_PER_W // TCHUNK, chunk, 0)

    return _sc_combine


def kernel(x, ln_scale, ln_bias, Wr, br, W1, b1, W2, b2):
    i0, i1, w0, w1 = _stage1(x, ln_scale, ln_bias, Wr, br)
    gather_idx, wg, blk_expert, posa, posb = _routing_metadata(i0, i1, w0, w1)
    xg = _sc_gather_fn()(x, gather_idx)
    rows = _ffn(blk_expert, xg, W1, b1, W2, b2, wg, ln_scale, ln_bias)
    y = _sc_combine_fn()(rows, posa, posb)
    return y
